# baseline (speedup 1.0000x reference)
import functools

import jax
import jax.numpy as jnp
from jax import lax
from jax.experimental import pallas as pl
from jax.experimental.pallas import tpu as pltpu
from jax.experimental.pallas import tpu_sc as plsc

E = 16
K = 2
H = 768
F = 3072
N = 2048
A = N * K
BLK = 512
NBLK = A // BLK + E
P = NBLK * BLK
NW = 32
CH = A // NW
RC = 512


def _router_kernel(hs_ref, rw_ref, pos_ref, probs_ref, be_ref, o_scr):
    hs = hs_ref[...]
    rw = rw_ref[...]
    logits = lax.dot_general(hs, rw, (((1,), (1,)), ((), ())),
                             preferred_element_type=jnp.float32)
    lane = lax.broadcasted_iota(jnp.int32, (N, E), 1).astype(jnp.float32)
    m0 = jnp.max(logits, axis=1, keepdims=True)
    idx0 = jnp.min(jnp.where(logits == m0, lane, 1e9), axis=1, keepdims=True)
    logits1 = jnp.where(lane == idx0, -1e30, logits)
    m1 = jnp.max(logits1, axis=1, keepdims=True)
    idx1 = jnp.min(jnp.where(logits1 == m1, lane, 1e9), axis=1, keepdims=True)

    ed = jnp.exp(m1 - m0)
    p0 = 1.0 / (1.0 + ed)
    probs_ref[...] = jnp.concatenate([p0, 1.0 - p0], axis=1)

    o0 = (lane == idx0).astype(jnp.float32)
    o1 = (lane == idx1).astype(jnp.float32)
    o_scr[0:N, :] = o0
    o_scr[N:A, :] = o1

    counts = jnp.round(jnp.sum(o0, axis=0, keepdims=True)
                       + jnp.sum(o1, axis=0, keepdims=True))
    nblk_e = jnp.floor((counts + (BLK - 1)) / BLK)
    lt = (lax.broadcasted_iota(jnp.int32, (E, E), 0)
          < lax.broadcasted_iota(jnp.int32, (E, E), 1)).astype(jnp.float32)
    le = (lax.broadcasted_iota(jnp.int32, (E, E), 0)
          <= lax.broadcasted_iota(jnp.int32, (E, E), 1)).astype(jnp.float32)
    pad_start = jnp.round(lax.dot_general(
        nblk_e, lt, (((1,), (0,)), ((), ())),
        preferred_element_type=jnp.float32)) * BLK
    cum_incl = jnp.round(lax.dot_general(
        nblk_e, le, (((1,), (0,)), ((), ())),
        preferred_element_type=jnp.float32))
    bi = lax.broadcasted_iota(jnp.int32, (NBLK + 8, E), 0).astype(jnp.float32)
    be = jnp.sum((bi >= cum_incl).astype(jnp.float32), axis=1, keepdims=True)
    be = jnp.minimum(be, E - 1).astype(jnp.int32)
    nused = cum_incl[0, E - 1].astype(jnp.int32)
    rowi = lax.broadcasted_iota(jnp.int32, (NBLK + 8, 1), 0)
    be_ref[...] = jnp.where(rowi == NBLK, nused, be)

    tri = (lax.broadcasted_iota(jnp.int32, (RC, RC), 0)
           > lax.broadcasted_iota(jnp.int32, (RC, RC), 1)).astype(jnp.bfloat16)

    def body(c, base):
        oc = o_scr[pl.ds(c * RC, RC), :]
        rc = jnp.round(lax.dot_general(tri, oc.astype(jnp.bfloat16),
                                       (((1,), (0,)), ((), ())),
                                       preferred_element_type=jnp.float32))
        dest = jnp.sum(oc * (rc + base + pad_start), axis=1, keepdims=True)
        pos_ref[pl.ds(c * RC, RC), :] = (dest + 0.5).astype(jnp.int32)
        return base + jnp.sum(oc, axis=0, keepdims=True)

    lax.fori_loop(0, A // RC, body, jnp.zeros((1, E), jnp.float32))


NW1 = 8
NW2 = 8


def _mlp_body(be_ref, x_ref, *rest):
    w1rs = rest[:NW1]
    b1_ref = rest[NW1]
    w2rs = rest[NW1 + 1:NW1 + 1 + NW2]
    b2_ref = rest[NW1 + 1 + NW2]
    y_ref = rest[NW1 + 2 + NW2]
    i = pl.program_id(0)

    @pl.when(i < be_ref[NBLK])
    def _():
        x = x_ref[...].astype(jnp.bfloat16)
        b1 = b1_ref[0]
        half = jnp.bfloat16(0.5)
        one = jnp.bfloat16(1.0)
        isq2 = jnp.bfloat16(0.70703125)
        fc = F // NW1
        hs_parts = []
        for j, wr in enumerate(w1rs):
            hc = lax.dot_general(x, wr[0, 0].astype(jnp.bfloat16),
                                 (((1,), (1,)), ((), ())),
                                 preferred_element_type=jnp.float32)
            hc = (hc + lax.slice(b1, (0, j * fc),
                                 (1, (j + 1) * fc))).astype(jnp.bfloat16)
            hc = half * hc * (one + lax.erf(hc * isq2))
            hs_parts.append(hc)
        h = jnp.concatenate(hs_parts, axis=1)
        ys = [lax.dot_general(h, wr[0, 0].astype(jnp.bfloat16),
                              (((1,), (1,)), ((), ())),
                              preferred_element_type=jnp.float32)
              for wr in w2rs]
        y_ref[...] = jnp.concatenate(ys, axis=1) + b2_ref[0]


def _combine_kernel(g_ref, p_ref, o_ref):
    p0 = p_ref[:, 0:1]
    p1 = p_ref[:, 1:2]
    g = g_ref[...]
    o_ref[...] = p0 * g[0:N, :] + p1 * g[N:A, :]


HCH = CH // 2


def _dispatch(hs2, pos_rows):
    mesh = plsc.VectorSubcoreMesh(core_axis_name="c", subcore_axis_name="s")

    @functools.partial(
        pl.kernel, mesh=mesh,
        out_type=jax.ShapeDtypeStruct((P, H), jnp.float32),
        scratch_types=[pltpu.VMEM((2, HCH), jnp.int32),
                       pltpu.VMEM((HCH, H), jnp.float32),
                       pltpu.VMEM((HCH, H), jnp.float32),
                       pltpu.SemaphoreType.DMA, pltpu.SemaphoreType.DMA,
                       pltpu.SemaphoreType.DMA, pltpu.SemaphoreType.DMA])
    def k(hs_hbm, pos_hbm, xpad_hbm, idx_v, rows0, rows1, s0, s1, s2, s3):
        wid = lax.axis_index("s") * 2 + lax.axis_index("c")
        base = wid * CH
        pltpu.sync_copy(pos_hbm.at[pl.ds(wid * 2, 2)], idx_v)
        src = lax.rem(base, N)
        in0 = pltpu.async_copy(hs_hbm.at[pl.ds(src, HCH), :], rows0, s0)
        in1 = pltpu.async_copy(hs_hbm.at[pl.ds(src + HCH, HCH), :], rows1, s1)
        in0.wait()
        out0 = pltpu.async_copy(rows0, xpad_hbm.at[idx_v.at[0]], s2)
        in1.wait()
        out1 = pltpu.async_copy(rows1, xpad_hbm.at[idx_v.at[1]], s3)
        out0.wait()
        out1.wait()

    return k(hs2, pos_rows)


def _unpermute(y, pos_rows):
    mesh = plsc.VectorSubcoreMesh(core_axis_name="c", subcore_axis_name="s")

    @functools.partial(
        pl.kernel, mesh=mesh,
        out_type=jax.ShapeDtypeStruct((A, H), jnp.float32),
        scratch_types=[pltpu.VMEM((2, HCH), jnp.int32),
                       pltpu.VMEM((HCH, H), jnp.float32),
                       pltpu.VMEM((HCH, H), jnp.float32),
                       pltpu.SemaphoreType.DMA, pltpu.SemaphoreType.DMA,
                       pltpu.SemaphoreType.DMA, pltpu.SemaphoreType.DMA])
    def k(y_hbm, pos_hbm, g_hbm, idx_v, rows0, rows1, s0, s1, s2, s3):
        wid = lax.axis_index("s") * 2 + lax.axis_index("c")
        base = wid * CH
        pltpu.sync_copy(pos_hbm.at[pl.ds(wid * 2, 2)], idx_v)
        in0 = pltpu.async_copy(y_hbm.at[idx_v.at[0]], rows0, s0)
        in1 = pltpu.async_copy(y_hbm.at[idx_v.at[1]], rows1, s1)
        in0.wait()
        out0 = pltpu.async_copy(rows0, g_hbm.at[pl.ds(base, HCH), :], s2)
        in1.wait()
        out1 = pltpu.async_copy(rows1, g_hbm.at[pl.ds(base + HCH, HCH), :], s3)
        out0.wait()
        out1.wait()

    return k(y, pos_rows)


def kernel(hidden_states, router_w, w1, b1, w2, b2):
    hidden_shape = hidden_states.shape
    hs2 = hidden_states.reshape(N, H)

    pos, probs, be = pl.pallas_call(
        _router_kernel,
        out_shape=(jax.ShapeDtypeStruct((A, 1), jnp.int32),
                   jax.ShapeDtypeStruct((N, K), jnp.float32),
                   jax.ShapeDtypeStruct((NBLK + 8, 1), jnp.int32)),
        scratch_shapes=[pltpu.VMEM((A, E), jnp.float32)],
    )(hs2, router_w)

    xpad = _dispatch(hs2, pos.reshape(2 * NW, HCH))

    def _wspec(j):
        return pl.BlockSpec((1, 1, F // NW1, H),
                            lambda i, be_s, j=j: (be_s[i], j, 0, 0))

    def _w2spec(j):
        return pl.BlockSpec((1, 1, H // NW2, F),
                            lambda i, be_s, j=j: (be_s[i], j, 0, 0))

    grid_spec = pltpu.PrefetchScalarGridSpec(
        num_scalar_prefetch=1,
        grid=(NBLK,),
        in_specs=[
            pl.BlockSpec((BLK, H),
                         lambda i, be_s: (jnp.minimum(i, be_s[NBLK] - 1), 0)),
            *[_wspec(j) for j in range(NW1)],
            pl.BlockSpec((1, 1, F), lambda i, be_s: (be_s[i], 0, 0)),
            *[_w2spec(j) for j in range(NW2)],
            pl.BlockSpec((1, 1, H), lambda i, be_s: (be_s[i], 0, 0)),
        ],
        out_specs=pl.BlockSpec(
            (BLK, H),
            lambda i, be_s: (jnp.where(i < be_s[NBLK], i, NBLK), 0)),
    )
    w1s = w1.reshape(E, NW1, F // NW1, H)
    w2s = w2.reshape(E, NW2, H // NW2, F)
    y = pl.pallas_call(
        _mlp_body,
        grid_spec=grid_spec,
        out_shape=jax.ShapeDtypeStruct((P + BLK, H), jnp.float32),
        compiler_params=pltpu.CompilerParams(
            dimension_semantics=("arbitrary",)),
    )(be.reshape(NBLK + 8), xpad, *([w1s] * NW1), b1.reshape(E, 1, F),
      *([w2s] * NW2), b2.reshape(E, 1, H))

    g = _unpermute(y, pos.reshape(2 * NW, HCH))

    out = pl.pallas_call(
        _combine_kernel,
        out_shape=jax.ShapeDtypeStruct((N, H), jnp.float32),
    )(g, probs)
    return out.reshape(hidden_shape)


# speedup vs baseline: 1.3159x; 1.3159x over previous
import functools

import jax
import jax.numpy as jnp
from jax import lax
from jax.experimental import pallas as pl
from jax.experimental.pallas import tpu as pltpu
from jax.experimental.pallas import tpu_sc as plsc

E = 16
K = 2
H = 768
F = 3072
N = 2048
A = N * K
BLK = 512
NBLK = A // BLK + E
P = NBLK * BLK
NW = 32
CH = A // NW
RC = 512


def _router_kernel(hs_ref, rw_ref, pos_ref, probs_ref, be_ref, o_scr):
    hs = hs_ref[...]
    rw = rw_ref[...]
    logits = lax.dot_general(hs, rw, (((1,), (1,)), ((), ())),
                             preferred_element_type=jnp.float32)
    lane = lax.broadcasted_iota(jnp.int32, (N, E), 1).astype(jnp.float32)
    m0 = jnp.max(logits, axis=1, keepdims=True)
    idx0 = jnp.min(jnp.where(logits == m0, lane, 1e9), axis=1, keepdims=True)
    logits1 = jnp.where(lane == idx0, -1e30, logits)
    m1 = jnp.max(logits1, axis=1, keepdims=True)
    idx1 = jnp.min(jnp.where(logits1 == m1, lane, 1e9), axis=1, keepdims=True)

    ed = jnp.exp(m1 - m0)
    p0 = 1.0 / (1.0 + ed)
    probs_ref[...] = jnp.concatenate([p0, 1.0 - p0], axis=1)

    o0 = (lane == idx0).astype(jnp.float32)
    o1 = (lane == idx1).astype(jnp.float32)
    o_scr[0:N, :] = o0
    o_scr[N:A, :] = o1

    counts = jnp.round(jnp.sum(o0, axis=0, keepdims=True)
                       + jnp.sum(o1, axis=0, keepdims=True))
    nblk_e = jnp.floor((counts + (BLK - 1)) / BLK)
    lt = (lax.broadcasted_iota(jnp.int32, (E, E), 0)
          < lax.broadcasted_iota(jnp.int32, (E, E), 1)).astype(jnp.float32)
    le = (lax.broadcasted_iota(jnp.int32, (E, E), 0)
          <= lax.broadcasted_iota(jnp.int32, (E, E), 1)).astype(jnp.float32)
    pad_start = jnp.round(lax.dot_general(
        nblk_e, lt, (((1,), (0,)), ((), ())),
        preferred_element_type=jnp.float32)) * BLK
    cum_incl = jnp.round(lax.dot_general(
        nblk_e, le, (((1,), (0,)), ((), ())),
        preferred_element_type=jnp.float32))
    bi = lax.broadcasted_iota(jnp.int32, (NBLK + 8, E), 0).astype(jnp.float32)
    be = jnp.sum((bi >= cum_incl).astype(jnp.float32), axis=1, keepdims=True)
    be = jnp.minimum(be, E - 1).astype(jnp.int32)
    nused = cum_incl[0, E - 1].astype(jnp.int32)
    rowi = lax.broadcasted_iota(jnp.int32, (NBLK + 8, 1), 0)
    be_ref[...] = jnp.where(rowi == NBLK, nused, be)

    tri = (lax.broadcasted_iota(jnp.int32, (RC, RC), 0)
           > lax.broadcasted_iota(jnp.int32, (RC, RC), 1)).astype(jnp.bfloat16)

    def body(c, base):
        oc = o_scr[pl.ds(c * RC, RC), :]
        rc = jnp.round(lax.dot_general(tri, oc.astype(jnp.bfloat16),
                                       (((1,), (0,)), ((), ())),
                                       preferred_element_type=jnp.float32))
        dest = jnp.sum(oc * (rc + base + pad_start), axis=1, keepdims=True)
        pos_ref[pl.ds(c * RC, RC), :] = (dest + 0.5).astype(jnp.int32)
        return base + jnp.sum(oc, axis=0, keepdims=True)

    lax.fori_loop(0, A // RC, body, jnp.zeros((1, E), jnp.float32))


NW1 = 4
NW2 = 4


def _mlp_body(be_ref, x_ref, *rest):
    w1rs = rest[:NW1]
    b1_ref = rest[NW1]
    w2rs = rest[NW1 + 1:NW1 + 1 + NW2]
    b2_ref = rest[NW1 + 1 + NW2]
    y_ref = rest[NW1 + 2 + NW2]
    i = pl.program_id(0)

    @pl.when(i < be_ref[NBLK])
    def _():
        x = x_ref[...].astype(jnp.bfloat16)
        b1 = b1_ref[0]
        half = jnp.bfloat16(0.5)
        one = jnp.bfloat16(1.0)
        isq2 = jnp.bfloat16(0.70703125)
        fc = F // NW1
        hs_parts = []
        for j, wr in enumerate(w1rs):
            hc = lax.dot_general(x, wr[0, 0].astype(jnp.bfloat16),
                                 (((1,), (1,)), ((), ())),
                                 preferred_element_type=jnp.float32)
            hc = (hc + lax.slice(b1, (0, j * fc),
                                 (1, (j + 1) * fc))).astype(jnp.bfloat16)
            hc = half * hc * (one + lax.erf(hc * isq2))
            hs_parts.append(hc)
        h = jnp.concatenate(hs_parts, axis=1)
        ys = [lax.dot_general(h, wr[0, 0].astype(jnp.bfloat16),
                              (((1,), (1,)), ((), ())),
                              preferred_element_type=jnp.float32)
              for wr in w2rs]
        y_ref[...] = jnp.concatenate(ys, axis=1) + b2_ref[0]


def _combine_kernel(g_ref, p_ref, o_ref):
    p0 = p_ref[:, 0:1]
    p1 = p_ref[:, 1:2]
    g = g_ref[...]
    o_ref[...] = p0 * g[0:N, :] + p1 * g[N:A, :]


HCH = CH // 2


def _dispatch(hs2, pos_rows):
    mesh = plsc.VectorSubcoreMesh(core_axis_name="c", subcore_axis_name="s")

    @functools.partial(
        pl.kernel, mesh=mesh,
        out_type=jax.ShapeDtypeStruct((P, H), jnp.float32),
        scratch_types=[pltpu.VMEM((2, HCH), jnp.int32),
                       pltpu.VMEM((HCH, H), jnp.float32),
                       pltpu.VMEM((HCH, H), jnp.float32),
                       pltpu.SemaphoreType.DMA, pltpu.SemaphoreType.DMA,
                       pltpu.SemaphoreType.DMA, pltpu.SemaphoreType.DMA])
    def k(hs_hbm, pos_hbm, xpad_hbm, idx_v, rows0, rows1, s0, s1, s2, s3):
        wid = lax.axis_index("s") * 2 + lax.axis_index("c")
        base = wid * CH
        pltpu.sync_copy(pos_hbm.at[pl.ds(wid * 2, 2)], idx_v)
        src = lax.rem(base, N)
        in0 = pltpu.async_copy(hs_hbm.at[pl.ds(src, HCH), :], rows0, s0)
        in1 = pltpu.async_copy(hs_hbm.at[pl.ds(src + HCH, HCH), :], rows1, s1)
        in0.wait()
        out0 = pltpu.async_copy(rows0, xpad_hbm.at[idx_v.at[0]], s2)
        in1.wait()
        out1 = pltpu.async_copy(rows1, xpad_hbm.at[idx_v.at[1]], s3)
        out0.wait()
        out1.wait()

    return k(hs2, pos_rows)


def _unpermute(y, pos_rows):
    mesh = plsc.VectorSubcoreMesh(core_axis_name="c", subcore_axis_name="s")

    @functools.partial(
        pl.kernel, mesh=mesh,
        out_type=jax.ShapeDtypeStruct((A, H), jnp.float32),
        scratch_types=[pltpu.VMEM((2, HCH), jnp.int32),
                       pltpu.VMEM((HCH, H), jnp.float32),
                       pltpu.VMEM((HCH, H), jnp.float32),
                       pltpu.SemaphoreType.DMA, pltpu.SemaphoreType.DMA,
                       pltpu.SemaphoreType.DMA, pltpu.SemaphoreType.DMA])
    def k(y_hbm, pos_hbm, g_hbm, idx_v, rows0, rows1, s0, s1, s2, s3):
        wid = lax.axis_index("s") * 2 + lax.axis_index("c")
        base = wid * CH
        pltpu.sync_copy(pos_hbm.at[pl.ds(wid * 2, 2)], idx_v)
        in0 = pltpu.async_copy(y_hbm.at[idx_v.at[0]], rows0, s0)
        in1 = pltpu.async_copy(y_hbm.at[idx_v.at[1]], rows1, s1)
        in0.wait()
        out0 = pltpu.async_copy(rows0, g_hbm.at[pl.ds(base, HCH), :], s2)
        in1.wait()
        out1 = pltpu.async_copy(rows1, g_hbm.at[pl.ds(base + HCH, HCH), :], s3)
        out0.wait()
        out1.wait()

    return k(y, pos_rows)


def kernel(hidden_states, router_w, w1, b1, w2, b2):
    hidden_shape = hidden_states.shape
    hs2 = hidden_states.reshape(N, H)

    pos, probs, be = pl.pallas_call(
        _router_kernel,
        out_shape=(jax.ShapeDtypeStruct((A, 1), jnp.int32),
                   jax.ShapeDtypeStruct((N, K), jnp.float32),
                   jax.ShapeDtypeStruct((NBLK + 8, 1), jnp.int32)),
        scratch_shapes=[pltpu.VMEM((A, E), jnp.float32)],
    )(hs2, router_w)

    xpad = _dispatch(hs2, pos.reshape(2 * NW, HCH))

    def _wspec(j):
        return pl.BlockSpec((1, 1, F // NW1, H),
                            lambda i, be_s, j=j: (be_s[i], j, 0, 0))

    def _w2spec(j):
        return pl.BlockSpec((1, 1, H // NW2, F),
                            lambda i, be_s, j=j: (be_s[i], j, 0, 0))

    grid_spec = pltpu.PrefetchScalarGridSpec(
        num_scalar_prefetch=1,
        grid=(NBLK,),
        in_specs=[
            pl.BlockSpec((BLK, H),
                         lambda i, be_s: (jnp.minimum(i, be_s[NBLK] - 1), 0)),
            *[_wspec(j) for j in range(NW1)],
            pl.BlockSpec((1, 1, F), lambda i, be_s: (be_s[i], 0, 0)),
            *[_w2spec(j) for j in range(NW2)],
            pl.BlockSpec((1, 1, H), lambda i, be_s: (be_s[i], 0, 0)),
        ],
        out_specs=pl.BlockSpec(
            (BLK, H),
            lambda i, be_s: (jnp.where(i < be_s[NBLK], i, NBLK), 0)),
    )
    w1s = w1.reshape(E, NW1, F // NW1, H)
    w2s = w2.reshape(E, NW2, H // NW2, F)
    y = pl.pallas_call(
        _mlp_body,
        grid_spec=grid_spec,
        out_shape=jax.ShapeDtypeStruct((P + BLK, H), jnp.float32),
        compiler_params=pltpu.CompilerParams(
            dimension_semantics=("arbitrary",)),
    )(be.reshape(NBLK + 8), xpad, *([w1s] * NW1), b1.reshape(E, 1, F),
      *([w2s] * NW2), b2.reshape(E, 1, H))

    g = _unpermute(y, pos.reshape(2 * NW, HCH))

    out = pl.pallas_call(
        _combine_kernel,
        out_shape=jax.ShapeDtypeStruct((N, H), jnp.float32),
    )(g, probs)
    return out.reshape(hidden_shape)


# speedup vs baseline: 1.3230x; 1.0053x over previous
import functools

import jax
import jax.numpy as jnp
from jax import lax
from jax.experimental import pallas as pl
from jax.experimental.pallas import tpu as pltpu
from jax.experimental.pallas import tpu_sc as plsc

E = 16
K = 2
H = 768
F = 3072
N = 2048
A = N * K
BLK = 512
NBLK = A // BLK + E
P = NBLK * BLK
NW = 32
CH = A // NW
RC = 512


def _router_kernel(h0, h1, h2, h3, rw_ref, pos_ref, probs_ref, be_ref, o_scr):
    rw = rw_ref[...]
    logits = jnp.concatenate(
        [lax.dot_general(hr[0], rw, (((1,), (1,)), ((), ())),
                         preferred_element_type=jnp.float32)
         for hr in (h0, h1, h2, h3)], axis=0)
    lane = lax.broadcasted_iota(jnp.int32, (N, E), 1).astype(jnp.float32)
    m0 = jnp.max(logits, axis=1, keepdims=True)
    idx0 = jnp.min(jnp.where(logits == m0, lane, 1e9), axis=1, keepdims=True)
    logits1 = jnp.where(lane == idx0, -1e30, logits)
    m1 = jnp.max(logits1, axis=1, keepdims=True)
    idx1 = jnp.min(jnp.where(logits1 == m1, lane, 1e9), axis=1, keepdims=True)

    ed = jnp.exp(m1 - m0)
    p0 = 1.0 / (1.0 + ed)
    probs_ref[...] = jnp.concatenate([p0, 1.0 - p0], axis=1)

    o0 = (lane == idx0).astype(jnp.float32)
    o1 = (lane == idx1).astype(jnp.float32)
    o_scr[0:N, :] = o0
    o_scr[N:A, :] = o1

    counts = jnp.round(jnp.sum(o0, axis=0, keepdims=True)
                       + jnp.sum(o1, axis=0, keepdims=True))
    nblk_e = jnp.floor((counts + (BLK - 1)) / BLK)
    lt = (lax.broadcasted_iota(jnp.int32, (E, E), 0)
          < lax.broadcasted_iota(jnp.int32, (E, E), 1)).astype(jnp.float32)
    le = (lax.broadcasted_iota(jnp.int32, (E, E), 0)
          <= lax.broadcasted_iota(jnp.int32, (E, E), 1)).astype(jnp.float32)
    pad_start = jnp.round(lax.dot_general(
        nblk_e, lt, (((1,), (0,)), ((), ())),
        preferred_element_type=jnp.float32)) * BLK
    cum_incl = jnp.round(lax.dot_general(
        nblk_e, le, (((1,), (0,)), ((), ())),
        preferred_element_type=jnp.float32))
    bi = lax.broadcasted_iota(jnp.int32, (NBLK + 8, E), 0).astype(jnp.float32)
    be = jnp.sum((bi >= cum_incl).astype(jnp.float32), axis=1, keepdims=True)
    be = jnp.minimum(be, E - 1).astype(jnp.int32)
    nused = cum_incl[0, E - 1].astype(jnp.int32)
    rowi = lax.broadcasted_iota(jnp.int32, (NBLK + 8, 1), 0)
    be_ref[...] = jnp.where(rowi == NBLK, nused, be)

    tri = (lax.broadcasted_iota(jnp.int32, (RC, RC), 0)
           > lax.broadcasted_iota(jnp.int32, (RC, RC), 1)).astype(jnp.bfloat16)

    def body(c, base):
        oc = o_scr[pl.ds(c * RC, RC), :]
        rc = jnp.round(lax.dot_general(tri, oc.astype(jnp.bfloat16),
                                       (((1,), (0,)), ((), ())),
                                       preferred_element_type=jnp.float32))
        dest = jnp.sum(oc * (rc + base + pad_start), axis=1, keepdims=True)
        pos_ref[pl.ds(c * RC, RC), :] = (dest + 0.5).astype(jnp.int32)
        return base + jnp.sum(oc, axis=0, keepdims=True)

    lax.fori_loop(0, A // RC, body, jnp.zeros((1, E), jnp.float32))


NW1 = 4
NW2 = 4


def _mlp_body(be_ref, x_ref, *rest):
    w1rs = rest[:NW1]
    b1_ref = rest[NW1]
    w2rs = rest[NW1 + 1:NW1 + 1 + NW2]
    b2_ref = rest[NW1 + 1 + NW2]
    y_ref = rest[NW1 + 2 + NW2]
    i = pl.program_id(0)

    @pl.when(i < be_ref[NBLK])
    def _():
        x = x_ref[...].astype(jnp.bfloat16)
        b1 = b1_ref[0]
        half = jnp.bfloat16(0.5)
        one = jnp.bfloat16(1.0)
        isq2 = jnp.bfloat16(0.70703125)
        fc = F // NW1
        hs_parts = []
        for j, wr in enumerate(w1rs):
            hc = lax.dot_general(x, wr[0, 0].astype(jnp.bfloat16),
                                 (((1,), (1,)), ((), ())),
                                 preferred_element_type=jnp.float32)
            hc = (hc + lax.slice(b1, (0, j * fc),
                                 (1, (j + 1) * fc))).astype(jnp.bfloat16)
            hc = half * hc * (one + lax.erf(hc * isq2))
            hs_parts.append(hc)
        h = jnp.concatenate(hs_parts, axis=1)
        ys = [lax.dot_general(h, wr[0, 0].astype(jnp.bfloat16),
                              (((1,), (1,)), ((), ())),
                              preferred_element_type=jnp.float32)
              for wr in w2rs]
        y_ref[...] = jnp.concatenate(ys, axis=1) + b2_ref[0]


def _combine_kernel(g_ref, p_ref, o_ref):
    p0 = p_ref[:, 0:1]
    p1 = p_ref[:, 1:2]
    g = g_ref[...]
    o_ref[...] = p0 * g[0:N, :] + p1 * g[N:A, :]


HCH = CH // 2


def _dispatch(hs2, pos_rows):
    mesh = plsc.VectorSubcoreMesh(core_axis_name="c", subcore_axis_name="s")

    @functools.partial(
        pl.kernel, mesh=mesh,
        out_type=jax.ShapeDtypeStruct((P, H), jnp.float32),
        scratch_types=[pltpu.VMEM((2, HCH), jnp.int32),
                       pltpu.VMEM((HCH, H), jnp.float32),
                       pltpu.VMEM((HCH, H), jnp.float32),
                       pltpu.SemaphoreType.DMA, pltpu.SemaphoreType.DMA,
                       pltpu.SemaphoreType.DMA, pltpu.SemaphoreType.DMA])
    def k(hs_hbm, pos_hbm, xpad_hbm, idx_v, rows0, rows1, s0, s1, s2, s3):
        wid = lax.axis_index("s") * 2 + lax.axis_index("c")
        base = wid * CH
        pltpu.sync_copy(pos_hbm.at[pl.ds(wid * 2, 2)], idx_v)
        src = lax.rem(base, N)
        in0 = pltpu.async_copy(hs_hbm.at[pl.ds(src, HCH), :], rows0, s0)
        in1 = pltpu.async_copy(hs_hbm.at[pl.ds(src + HCH, HCH), :], rows1, s1)
        in0.wait()
        out0 = pltpu.async_copy(rows0, xpad_hbm.at[idx_v.at[0]], s2)
        in1.wait()
        out1 = pltpu.async_copy(rows1, xpad_hbm.at[idx_v.at[1]], s3)
        out0.wait()
        out1.wait()

    return k(hs2, pos_rows)


def _unpermute(y, pos_rows):
    mesh = plsc.VectorSubcoreMesh(core_axis_name="c", subcore_axis_name="s")

    @functools.partial(
        pl.kernel, mesh=mesh,
        out_type=jax.ShapeDtypeStruct((A, H), jnp.float32),
        scratch_types=[pltpu.VMEM((2, HCH), jnp.int32),
                       pltpu.VMEM((HCH, H), jnp.float32),
                       pltpu.VMEM((HCH, H), jnp.float32),
                       pltpu.SemaphoreType.DMA, pltpu.SemaphoreType.DMA,
                       pltpu.SemaphoreType.DMA, pltpu.SemaphoreType.DMA])
    def k(y_hbm, pos_hbm, g_hbm, idx_v, rows0, rows1, s0, s1, s2, s3):
        wid = lax.axis_index("s") * 2 + lax.axis_index("c")
        base = wid * CH
        pltpu.sync_copy(pos_hbm.at[pl.ds(wid * 2, 2)], idx_v)
        in0 = pltpu.async_copy(y_hbm.at[idx_v.at[0]], rows0, s0)
        in1 = pltpu.async_copy(y_hbm.at[idx_v.at[1]], rows1, s1)
        in0.wait()
        out0 = pltpu.async_copy(rows0, g_hbm.at[pl.ds(base, HCH), :], s2)
        in1.wait()
        out1 = pltpu.async_copy(rows1, g_hbm.at[pl.ds(base + HCH, HCH), :], s3)
        out0.wait()
        out1.wait()

    return k(y, pos_rows)


def kernel(hidden_states, router_w, w1, b1, w2, b2):
    hidden_shape = hidden_states.shape
    hs2 = hidden_states.reshape(N, H)

    hs4 = hs2.reshape(4, N // 4, H)
    pos, probs, be = pl.pallas_call(
        _router_kernel,
        out_shape=(jax.ShapeDtypeStruct((A, 1), jnp.int32),
                   jax.ShapeDtypeStruct((N, K), jnp.float32),
                   jax.ShapeDtypeStruct((NBLK + 8, 1), jnp.int32)),
        grid=(1,),
        in_specs=[
            pl.BlockSpec((1, N // 4, H), lambda i, j=j: (j, 0, 0))
            for j in range(4)
        ] + [pl.BlockSpec((E, H), lambda i: (0, 0))],
        out_specs=(pl.BlockSpec((A, 1), lambda i: (0, 0)),
                   pl.BlockSpec((N, K), lambda i: (0, 0)),
                   pl.BlockSpec((NBLK + 8, 1), lambda i: (0, 0))),
        scratch_shapes=[pltpu.VMEM((A, E), jnp.float32)],
    )(hs4, hs4, hs4, hs4, router_w)

    xpad = _dispatch(hs2, pos.reshape(2 * NW, HCH))

    def _wspec(j):
        return pl.BlockSpec((1, 1, F // NW1, H),
                            lambda i, be_s, j=j: (be_s[i], j, 0, 0))

    def _w2spec(j):
        return pl.BlockSpec((1, 1, H // NW2, F),
                            lambda i, be_s, j=j: (be_s[i], j, 0, 0))

    grid_spec = pltpu.PrefetchScalarGridSpec(
        num_scalar_prefetch=1,
        grid=(NBLK,),
        in_specs=[
            pl.BlockSpec((BLK, H),
                         lambda i, be_s: (jnp.minimum(i, be_s[NBLK] - 1), 0)),
            *[_wspec(j) for j in range(NW1)],
            pl.BlockSpec((1, 1, F), lambda i, be_s: (be_s[i], 0, 0)),
            *[_w2spec(j) for j in range(NW2)],
            pl.BlockSpec((1, 1, H), lambda i, be_s: (be_s[i], 0, 0)),
        ],
        out_specs=pl.BlockSpec(
            (BLK, H),
            lambda i, be_s: (jnp.where(i < be_s[NBLK], i, NBLK), 0)),
    )
    w1s = w1.reshape(E, NW1, F // NW1, H)
    w2s = w2.reshape(E, NW2, H // NW2, F)
    y = pl.pallas_call(
        _mlp_body,
        grid_spec=grid_spec,
        out_shape=jax.ShapeDtypeStruct((P + BLK, H), jnp.float32),
        compiler_params=pltpu.CompilerParams(
            dimension_semantics=("arbitrary",)),
    )(be.reshape(NBLK + 8), xpad, *([w1s] * NW1), b1.reshape(E, 1, F),
      *([w2s] * NW2), b2.reshape(E, 1, H))

    g = _unpermute(y, pos.reshape(2 * NW, HCH))

    out = pl.pallas_call(
        _combine_kernel,
        out_shape=jax.ShapeDtypeStruct((N, H), jnp.float32),
    )(g, probs)
    return out.reshape(hidden_shape)
